# baseline (speedup 1.0000x reference)
import functools

import numpy as np
import jax
import jax.numpy as jnp
from jax.experimental import pallas as pl

_MASK_PROB = 0.65
_MASK_LENGTH = 10
_MASK_SELECTION = "static"
_MASK_OTHER = 0.0
_NO_MASK_OVERLAP = False
_MASK_MIN_SPACE = 1
_MASK_CHANNEL_PROB = 0.1
_MASK_CHANNEL_LENGTH = 64
_MASK_CHANNEL_SELECTION = "static"
_MASK_CHANNEL_OTHER = 0.0
_NO_MASK_CHANNEL_OVERLAP = False
_MASK_CHANNEL_MIN_SPACE = 1


def _mask_indices_np(shape, padding_mask, mask_prob, mask_length, mask_type, mask_other, min_masks=0, no_overlap=False, min_space=0):
    bsz, all_sz = shape
    mask = np.full((bsz, all_sz), False)
    all_num_mask = int(mask_prob * all_sz / float(mask_length) + np.random.rand())
    all_num_mask = max(min_masks, all_num_mask)
    mask_idcs = []
    for i in range(bsz):
        if padding_mask is not None:
            sz = all_sz - int(padding_mask[i].sum())
            num_mask = int(mask_prob * sz / float(mask_length) + np.random.rand())
            num_mask = max(min_masks, num_mask)
        else:
            sz = all_sz
            num_mask = all_num_mask
        if mask_type == "static":
            lengths = np.full(num_mask, mask_length)
        elif mask_type == "uniform":
            lengths = np.random.randint(mask_other, mask_length * 2 + 1, size=num_mask)
        elif mask_type == "normal":
            lengths = np.random.normal(mask_length, mask_other, size=num_mask)
            lengths = np.asarray([max(1, int(round(x))) for x in lengths])
        elif mask_type == "poisson":
            lengths = np.random.poisson(mask_length, size=num_mask)
            lengths = np.asarray([int(round(x)) for x in lengths])
        else:
            raise Exception("unknown mask selection " + mask_type)
        if sum(lengths) == 0:
            lengths[0] = min(mask_length, sz - 1)
        if no_overlap:
            mask_idc = []

            def arrange(s, e, length, keep_length):
                span_start = np.random.randint(s, e - length)
                mask_idc.extend(span_start + j for j in range(length))
                new_parts = []
                if span_start - s - min_space >= keep_length:
                    new_parts.append((s, span_start - min_space + 1))
                if e - span_start - length - min_space > keep_length:
                    new_parts.append((span_start + length + min_space, e))
                return new_parts

            parts = [(0, sz)]
            min_length = min(lengths)
            for length in sorted(lengths, reverse=True):
                lens = np.fromiter((e - s if e - s >= length + min_space else 0 for s, e in parts), np.int_)
                l_sum = np.sum(lens)
                if l_sum == 0:
                    break
                probs = lens / np.sum(lens)
                c = np.random.choice(len(parts), p=probs)
                s, e = parts.pop(c)
                parts.extend(arrange(s, e, length, min_length))
            mask_idc = np.asarray(mask_idc)
        else:
            min_len = min(lengths)
            if sz - min_len <= num_mask:
                min_len = sz - num_mask - 1
            mask_idc = np.random.choice(sz - min_len, num_mask, replace=False)
            mask_idc = np.asarray([mask_idc[j] + offset for j in range(len(mask_idc)) for offset in range(lengths[j])])
        mask_idcs.append(np.unique(mask_idc[mask_idc < sz]))
    min_len = min([len(m) for m in mask_idcs])
    for i, mask_idc in enumerate(mask_idcs):
        if len(mask_idc) > min_len:
            mask_idc = np.random.choice(mask_idc, min_len, replace=False)
        mask[i, mask_idc] = True
    return mask


@functools.lru_cache(maxsize=None)
def _host_masks(B, T, C):
    np.random.seed(0)
    pm = np.zeros((B, T), dtype=bool)
    mt = _mask_indices_np((B, T), pm, _MASK_PROB, _MASK_LENGTH, _MASK_SELECTION,
                          _MASK_OTHER, min_masks=2, no_overlap=_NO_MASK_OVERLAP,
                          min_space=_MASK_MIN_SPACE)
    mc = _mask_indices_np((B, C), None, _MASK_CHANNEL_PROB, _MASK_CHANNEL_LENGTH,
                          _MASK_CHANNEL_SELECTION, _MASK_CHANNEL_OTHER,
                          no_overlap=_NO_MASK_CHANNEL_OVERLAP,
                          min_space=_MASK_CHANNEL_MIN_SPACE)
    return mt, mc


def _select_body(x_ref, m_ref, mult_ref, embm_ref, o_ref):
    xv = x_ref[0]
    m = m_ref[0, 0, 0][:, None] > 0
    mult = mult_ref[0, 0]
    embm = embm_ref[0, 0]
    o_ref[0] = jnp.where(m, embm, xv * mult)


def kernel(x, padding_mask, mask_embedding):
    B, T, C = x.shape
    mt_np, mc_np = _host_masks(B, T, C)
    mask_indices = jnp.asarray(mt_np)

    BT = 2048
    NTB = T // BT

    m_f = jnp.where(jnp.logical_and(mask_indices, jnp.logical_not(padding_mask)),
                    jnp.float32(1.0), jnp.float32(0.0))
    m_f = m_f.reshape(B, NTB, 1, BT)
    mult = jnp.asarray((~mc_np).astype(np.float32)).reshape(B, 1, C)
    embm = mult * mask_embedding.astype(jnp.float32)[None, None, :]

    out = pl.pallas_call(
        _select_body,
        grid=(B, NTB),
        in_specs=[
            pl.BlockSpec((1, BT, C), lambda b, t: (b, t, 0)),
            pl.BlockSpec((1, 1, 1, BT), lambda b, t: (b, t, 0, 0)),
            pl.BlockSpec((1, 1, C), lambda b, t: (b, 0, 0)),
            pl.BlockSpec((1, 1, C), lambda b, t: (b, 0, 0)),
        ],
        out_specs=pl.BlockSpec((1, BT, C), lambda b, t: (b, t, 0)),
        out_shape=jax.ShapeDtypeStruct((B, T, C), x.dtype),
    )(x, m_f, mult, embm)

    return (out, mask_indices)
